# baseline (speedup 1.0000x reference)
import functools

import jax
import jax.numpy as jnp
from jax import lax
from jax.experimental import pallas as pl
from jax.experimental.pallas import tpu as pltpu
from jax.experimental.pallas import tpu_sc as plsc

N = 10000
E = 160000
F = 256
FH = 128
NC = 2
NS = 16
L = 16
EPT = E // NS
CH = 80
NCH = EPT // CH
NBLK = N // CH
MAXB = (NBLK + NS - 1) // NS


def _tc_body(x_ref, w_ref, att_ref, wx2_ref, a_ref):
    xb = x_ref[...]
    wx = lax.dot_general(xb, w_ref[...], (((1,), (1,)), ((), ())),
                         preferred_element_type=jnp.float32)
    wx2_ref[0] = wx[:, :FH]
    wx2_ref[1] = wx[:, FH:]
    ab = jnp.dot(wx, att_ref[...], preferred_element_type=jnp.float32)
    a_ref[0] = ab[:, :L]
    a_ref[1] = ab[:, L:]


def _tc_matmul(x, W, att32):
    nb = 1000
    grid = N // nb
    return pl.pallas_call(
        _tc_body,
        grid=(grid,),
        in_specs=[
            pl.BlockSpec((nb, F), lambda i: (i, 0)),
            pl.BlockSpec((F, F), lambda i: (0, 0)),
            pl.BlockSpec((F, 2 * L), lambda i: (0, 0)),
        ],
        out_specs=[
            pl.BlockSpec((NC, nb, FH), lambda i: (0, i, 0)),
            pl.BlockSpec((2, nb, L), lambda i: (0, i, 0)),
        ],
        out_shape=[
            jax.ShapeDtypeStruct((NC, N, FH), jnp.float32),
            jax.ShapeDtypeStruct((2, N, L), jnp.float32),
        ],
    )(x, W, att32)


def _sc_body(wx_ref, asr_ref, adr_ref, eidx_ref, out_ref,
             sedge, wxidx, dscat, avs, avd, dv, expc, rows, coeff, qf,
             semA, semB, semC, semD, semE, semF, denom_sp, acc_sp):
    c = lax.axis_index("c")
    s = lax.axis_index("s")
    iota = lax.iota(jnp.int32, L)
    head_mask = iota < 4
    zero16 = jnp.zeros((L,), jnp.float32)
    ebase = pl.multiple_of(s * EPT, 8)
    coff = c * N

    def idx_load(j, sl):
        off = ebase + j * CH
        pltpu.sync_copy(eidx_ref.at[:, pl.ds(off, CH)], sedge.at[sl])

    def idx_issue(j, sl):
        off = ebase + j * CH
        pltpu.async_copy(eidx_ref.at[:, pl.ds(off, CH)], sedge.at[sl], semE)

    def idx_wait(j, sl):
        off = ebase + j * CH
        pltpu.make_async_copy(eidx_ref.at[:, pl.ds(off, CH)], sedge.at[sl],
                              semE).wait()

    def compute_wi(sl):
        for l in range(CH // L):
            wxidx[sl, pl.ds(l * L, L)] = sedge[sl, 0, pl.ds(l * L, L)] + coff

    def issue_p1(sl):
        return [
            pltpu.async_copy(asr_ref.at[sedge.at[sl, 0]], avs.at[sl], semA),
            pltpu.async_copy(adr_ref.at[sedge.at[sl, 1]], avd.at[sl], semB),
        ]

    def issue_p2(sl):
        return issue_p1(sl) + [
            pltpu.async_copy(wx_ref.at[wxidx.at[sl]], rows.at[sl], semC),
            pltpu.async_copy(denom_sp.at[sedge.at[sl, 1]], dv.at[sl], semD),
        ]

    def zero_bufs(k, carry):
        for l in range(FH // L):
            rows[0, k, pl.ds(l * L, L)] = zero16
        expc[0, k, :] = zero16
        return carry
    lax.fori_loop(0, CH, zero_bufs, 0)

    def z_blk(k, carry):
        b = s + k * NS

        @pl.when(b < NBLK)
        def _():
            nb0 = pl.multiple_of(b * CH, 8)
            pltpu.sync_copy(rows.at[0], acc_sp.at[pl.ds(nb0, CH), :])
            pltpu.sync_copy(expc.at[0], denom_sp.at[pl.ds(nb0, CH), :])
        return carry
    lax.fori_loop(0, MAXB, z_blk, 0)
    plsc.subcore_barrier()

    idx_load(0, 0)
    for d in issue_p1(0):
        d.wait()
    idx_issue(1, 1)

    def p1_body(j, carry):
        p = j & 1
        jn = jnp.minimum(j + 1, NCH - 1)
        jn2 = jnp.minimum(j + 2, NCH - 1)
        idx_wait(jn, 1 - p)
        descs = issue_p1(1 - p)
        for l in range(CH // L):
            dscat[p, pl.ds(l * L, L)] = sedge[p, 1, pl.ds(l * L, L)]

        @plsc.parallel_loop(0, CH, unroll=4)
        def _(k):
            ee = avs[p, k, :] + avd[p, k, :]
            ee = jnp.maximum(ee, 0.2 * ee)
            expc[p, k, :] = jnp.where(head_mask, jnp.exp(ee), 0.0)

        @pl.when(j > 0)
        def _():
            pltpu.make_async_copy(expc.at[1 - p],
                                  denom_sp.at[dscat.at[1 - p]], semF).wait()
        pltpu.async_copy(expc.at[p], denom_sp.at[dscat.at[p]], semF,
                         add=True)
        idx_issue(jn2, p)
        for d in descs:
            d.wait()
        return carry
    lax.fori_loop(0, NCH, p1_body, 0)
    idx_wait(NCH - 1, (NCH - 1) & 1)
    pf = (NCH - 1) & 1
    pltpu.make_async_copy(expc.at[pf], denom_sp.at[dscat.at[pf]],
                          semF).wait()
    plsc.subcore_barrier()

    def r_blk(k, carry):
        b = s + k * NS

        @pl.when(b < NBLK)
        def _():
            nb0 = pl.multiple_of(b * CH, 8)
            pltpu.sync_copy(denom_sp.at[pl.ds(nb0, CH), :], expc.at[0])

            @plsc.parallel_loop(0, CH, unroll=4)
            def _(r):
                expc[0, r, :] = 1.0 / (expc[0, r, :] + 2e-9)
            pltpu.sync_copy(expc.at[0], denom_sp.at[pl.ds(nb0, CH), :])
        return carry
    lax.fori_loop(0, MAXB, r_blk, 0)
    plsc.subcore_barrier()

    idx_load(0, 0)
    compute_wi(0)
    for d in issue_p2(0):
        d.wait()
    idx_issue(1, 1)

    def p2_body(j, carry):
        p = j & 1
        jn = jnp.minimum(j + 1, NCH - 1)
        jn2 = jnp.minimum(j + 2, NCH - 1)
        idx_wait(jn, 1 - p)
        compute_wi(1 - p)
        descs = issue_p2(1 - p)

        @plsc.parallel_loop(0, CH, unroll=4)
        def _(k):
            ee = avs[p, k, :] + avd[p, k, :]
            ee = jnp.maximum(ee, 0.2 * ee)
            qf[pl.ds(k * L, L)] = jnp.exp(ee) * dv[p, k, :]

        @plsc.parallel_loop(0, CH // L, unroll=5)
        def _(m):
            i0 = (m * L + iota) * L
            v = (plsc.load_gather(qf, [i0])
                 + plsc.load_gather(qf, [i0 + 1])
                 + plsc.load_gather(qf, [i0 + 2])
                 + plsc.load_gather(qf, [i0 + 3]))
            coeff[pl.ds(m * L, L)] = v

        @plsc.parallel_loop(0, CH, unroll=4)
        def _(k):
            cb = plsc.load_gather(coeff, [jnp.zeros((L,), jnp.int32) + k])
            for l in range(FH // L):
                rows[p, k, pl.ds(l * L, L)] = (
                    rows[p, k, pl.ds(l * L, L)] * cb)
        pltpu.sync_copy(rows.at[p], acc_sp.at[sedge.at[p, 1]], add=True)
        idx_issue(jn2, p)
        for d in descs:
            d.wait()
        return carry
    lax.fori_loop(0, NCH, p2_body, 0)
    idx_wait(NCH - 1, (NCH - 1) & 1)
    plsc.subcore_barrier()

    cfh = pl.multiple_of(c * FH, FH)

    def p3_blk(k, carry):
        b = s + k * NS

        @pl.when(b < NBLK)
        def _():
            nb0 = pl.multiple_of(b * CH, 8)
            pltpu.sync_copy(acc_sp.at[pl.ds(nb0, CH), :], rows.at[0])

            @plsc.parallel_loop(0, CH, unroll=2)
            def _(r):
                for l in range(FH // L):
                    v = rows[0, r, pl.ds(l * L, L)] * 0.25
                    rows[0, r, pl.ds(l * L, L)] = jnp.where(
                        v > 0, v, jnp.exp(v) - 1.0)
            pltpu.sync_copy(rows.at[0],
                            out_ref.at[pl.ds(nb0, CH), pl.ds(cfh, FH)])
        return carry
    lax.fori_loop(0, MAXB, p3_blk, 0)


@functools.partial(
    pl.kernel,
    out_type=jax.ShapeDtypeStruct((N, F), jnp.float32),
    mesh=plsc.VectorSubcoreMesh(core_axis_name="c", subcore_axis_name="s"),
    scratch_types=[
        pltpu.VMEM((2, 2, CH), jnp.int32),
        pltpu.VMEM((2, CH), jnp.int32),
        pltpu.VMEM((2, CH), jnp.int32),
        pltpu.VMEM((2, CH, L), jnp.float32),
        pltpu.VMEM((2, CH, L), jnp.float32),
        pltpu.VMEM((2, CH, L), jnp.float32),
        pltpu.VMEM((2, CH, L), jnp.float32),
        pltpu.VMEM((2, CH, FH), jnp.float32),
        pltpu.VMEM((CH,), jnp.float32),
        pltpu.VMEM((CH * L,), jnp.float32),
        pltpu.SemaphoreType.DMA,
        pltpu.SemaphoreType.DMA,
        pltpu.SemaphoreType.DMA,
        pltpu.SemaphoreType.DMA,
        pltpu.SemaphoreType.DMA,
        pltpu.SemaphoreType.DMA,
        pltpu.VMEM_SHARED((N, L), jnp.float32),
        pltpu.VMEM_SHARED((N, FH), jnp.float32),
    ],
    compiler_params=pltpu.CompilerParams(needs_layout_passes=False,
                                         use_tc_tiling_on_sc=False),
)
def _sc_edge_kernel(wx_ref, asr_ref, adr_ref, eidx_ref, out_ref, *scratch):
    _sc_body(wx_ref, asr_ref, adr_ref, eidx_ref, out_ref, *scratch)


def kernel(x, edge_index, W, att_W):
    eidx = edge_index.astype(jnp.int32)
    att32 = jnp.zeros((F, 2 * L), jnp.float32)
    att32 = att32.at[:, 0:4].set(att_W[:, :F].T)
    att32 = att32.at[:, L:L + 4].set(att_W[:, F:].T)
    wx2, a2 = _tc_matmul(x, W, att32)
    wx_cat = wx2.reshape(NC * N, FH)
    return _sc_edge_kernel(wx_cat, a2[0], a2[1], eidx)
